# baseline (speedup 1.0000x reference)
import functools

import jax
import jax.numpy as jnp
from jax import lax
from jax.experimental import pallas as pl
from jax.experimental.pallas import tpu as pltpu
from jax.experimental.pallas import tpu_sc as plsc

E = 160000
D = 128
S = 5
R = 2 * S + 2
C = 20
GROWS = R * C
NCH = E // C
NW = 32
CPW = NCH // NW
NBUF = 3
VPR = D // 16


def _make_sc_kernel():
    mesh = plsc.VectorSubcoreMesh(core_axis_name="c", subcore_axis_name="s")

    @functools.partial(
        pl.kernel,
        mesh=mesh,
        out_type=(
            jax.ShapeDtypeStruct((NCH, 2 * C, D), jnp.float32),
            jax.ShapeDtypeStruct((NCH, 2 * C, D), jnp.float32),
        ),
        scratch_types=(
            [
                pltpu.VMEM((NBUF, 2, GROWS // 2), jnp.int32),
                pltpu.VMEM((NBUF, GROWS, D), jnp.float32),
                pltpu.VMEM((NBUF, 2 * C, D), jnp.float32),
            ]
            + [pltpu.SemaphoreType.DMA] * (3 * NBUF)
        ),
    )
    def k(x_hbm, idx_hbm, ea_hbm, nb_hbm, idxv, buf, nbo, *sems):
        gsem = sems[0:NBUF]
        isem = sems[NBUF:2 * NBUF]
        osem = sems[2 * NBUF:3 * NBUF]
        wid = lax.axis_index("s") * 2 + lax.axis_index("c")
        cbase = wid * CPW

        def fire_idx(j, b):
            pltpu.async_copy(idx_hbm.at[cbase + j], idxv.at[b], isem[b])

        def wait_idx(b):
            pltpu.make_async_copy(idx_hbm.at[cbase], idxv.at[b],
                                  isem[b]).wait()

        def fire_gather(b):
            for g in range(2):
                pltpu.async_copy(x_hbm.at[idxv.at[b, g]],
                                 buf.at[b, pl.ds(g * (GROWS // 2),
                                                 GROWS // 2)], gsem[b])

        def wait_gather(b):
            for g in range(2):
                pltpu.make_async_copy(
                    x_hbm.at[idxv.at[b, g]],
                    buf.at[b, pl.ds(g * (GROWS // 2), GROWS // 2)],
                    gsem[b]).wait()

        def fire_out(j, b):
            ch = cbase + j
            pltpu.async_copy(buf.at[b, pl.ds(0, 2 * C)], ea_hbm.at[ch],
                             osem[b])
            pltpu.async_copy(nbo.at[b], nb_hbm.at[ch], osem[b])

        def wait_out(b):
            for _ in range(2):
                pltpu.make_async_copy(nbo.at[b], nb_hbm.at[0],
                                      osem[b]).wait()

        def compute(b):
            def cbody(c, cc):
                for v in range(VPR):
                    sl = pl.ds(16 * v, 16)
                    s_ = buf[b, 2 * c, sl]
                    d_ = buf[b, 2 * c + 1, sl]
                    nbo[b, 2 * c, sl] = (s_ + d_) * 0.5
                    acc = buf[b, 2 * C + c, sl]
                    for r in range(1, 2 * S):
                        acc = acc + buf[b, 2 * C + r * C + c, sl]
                    nbo[b, 2 * c + 1, sl] = acc * jnp.float32(1.0 / (2 * S))
                return cc

            lax.fori_loop(0, C, cbody, 0)

        def do_slot(j, b, with_owait, with_fire):
            bn = (b + 1) % NBUF
            bi = (b + 2) % NBUF
            wait_gather(b)
            if with_owait:
                wait_out(bn)
            if with_fire:
                wait_idx(bn)
                fire_gather(bn)
                fire_idx(jnp.minimum(j + 2, CPW - 1), bi)
            compute(b)
            fire_out(j, b)

        pltpu.sync_copy(idx_hbm.at[cbase + 0], idxv.at[0])
        fire_idx(1, 1)
        fire_gather(0)

        do_slot(jnp.int32(0), 0, with_owait=False, with_fire=True)
        do_slot(jnp.int32(1), 1, with_owait=False, with_fire=True)

        def body(i, carry):
            jb = 3 * i + 2
            for u in range(3):
                do_slot(jb + u, (2 + u) % NBUF, with_owait=True,
                        with_fire=True)
            return carry

        lax.fori_loop(0, (CPW - 3) // 3, body, 0)

        do_slot(jnp.int32(CPW - 2), (CPW - 2) % NBUF, with_owait=True,
                with_fire=True)
        do_slot(jnp.int32(CPW - 1), (CPW - 1) % NBUF, with_owait=True,
                with_fire=False)

        wait_out((CPW - 2) % NBUF)
        wait_out((CPW - 1) % NBUF)
        wait_idx(CPW % NBUF)

    return k


_sc_agg = _make_sc_kernel()


def kernel(x, edge_index, nb_idx):
    src = edge_index[0]
    dst = edge_index[1]
    ed_inter = jnp.stack([src, dst], axis=-1).reshape(NCH, 2 * C)
    nb_part = (
        jnp.concatenate([jnp.transpose(nb_idx[0]),
                         jnp.transpose(nb_idx[1])], axis=0)
        .reshape(2 * S, NCH, C)
        .transpose(1, 0, 2)
        .reshape(NCH, 2 * S * C)
    )
    idx_ch = jnp.concatenate([ed_inter, nb_part], axis=1).reshape(
        NCH, 2, GROWS // 2)
    ea, nb = _sc_agg(x, idx_ch)
    return ea.reshape(E, 2 * D), nb.reshape(E, 2 * D)
